# Initial kernel scaffold; baseline (speedup 1.0000x reference)
import jax
import jax.numpy as jnp
from jax.experimental import pallas as pl


def kernel(x, embedding_table):
    raise NotImplementedError("write your pallas kernel here")


# speedup vs baseline: 2.0082x; 2.0082x over previous
import functools

import jax
import jax.numpy as jnp
from jax import lax
from jax.experimental import pallas as pl
from jax.experimental.pallas import tpu as pltpu
from jax.experimental.pallas import tpu_sc as plsc

NUM_ROWS = 20
EMB_DIM = 16
BATCH = 16384
PEP_LEN = 9
FLAT = BATCH * PEP_LEN

_info = plsc.get_sparse_core_info()
_NC, _NS = _info.num_cores, _info.num_subcores
_NW = _NC * _NS
_B_PER_W = FLAT // _NW


def _make_lookup():
  mesh = plsc.VectorSubcoreMesh(core_axis_name="c", subcore_axis_name="s")

  @functools.partial(
      pl.kernel,
      mesh=mesh,
      out_type=jax.ShapeDtypeStruct((FLAT, EMB_DIM), jnp.float32),
      scratch_types=[
          pltpu.VMEM((_B_PER_W,), jnp.int32),
          pltpu.VMEM((_B_PER_W, EMB_DIM), jnp.float32),
          pltpu.SemaphoreType.DMA,
      ],
      compiler_params=pltpu.CompilerParams(use_tc_tiling_on_sc=False),
  )
  def lookup(table_hbm, idx_hbm, out_hbm, idx_v, rows_v, sem):
    wid = lax.axis_index("s") * _NC + lax.axis_index("c")
    base = wid * _B_PER_W
    pltpu.sync_copy(idx_hbm.at[pl.ds(base, _B_PER_W)], idx_v)
    pltpu.async_copy(table_hbm.at[idx_v], rows_v, sem).wait()
    pltpu.sync_copy(rows_v, out_hbm.at[pl.ds(base, _B_PER_W)])

  return lookup


_lookup = _make_lookup()


@jax.jit
def kernel(x, embedding_table):
  idx = x.reshape(FLAT).astype(jnp.int32)
  out = _lookup(embedding_table, idx)
  return out.reshape(BATCH, PEP_LEN, EMB_DIM)


# speedup vs baseline: 4.6587x; 2.3199x over previous
import functools

import jax
import jax.numpy as jnp
from jax import lax
from jax.experimental import pallas as pl
from jax.experimental.pallas import tpu as pltpu
from jax.experimental.pallas import tpu_sc as plsc

NUM_ROWS = 20
EMB_DIM = 16
BATCH = 16384
PEP_LEN = 9
FLAT = BATCH * PEP_LEN

_info = plsc.get_sparse_core_info()
_NC, _NS, _NL = _info.num_cores, _info.num_subcores, _info.num_lanes
_NW = _NC * _NS
_B_PER_W = FLAT // _NW
_GROUPS = _B_PER_W // _NL


def _make_lookup():
  mesh = plsc.VectorSubcoreMesh(core_axis_name="c", subcore_axis_name="s")

  @functools.partial(
      pl.kernel,
      mesh=mesh,
      out_type=jax.ShapeDtypeStruct((FLAT, EMB_DIM), jnp.float32),
      scratch_types=[
          pltpu.VMEM((NUM_ROWS, EMB_DIM), jnp.float32),
          pltpu.VMEM((_B_PER_W,), jnp.int32),
          pltpu.VMEM((_B_PER_W, EMB_DIM), jnp.float32),
      ],
      compiler_params=pltpu.CompilerParams(
          use_tc_tiling_on_sc=False, needs_layout_passes=False
      ),
  )
  def lookup(table_hbm, idx_hbm, out_hbm, table_v, idx_v, rows_v):
    wid = lax.axis_index("s") * _NC + lax.axis_index("c")
    base = wid * _B_PER_W
    pltpu.sync_copy(table_hbm, table_v)
    pltpu.sync_copy(idx_hbm.at[pl.ds(base, _B_PER_W)], idx_v)

    lane = lax.iota(jnp.int32, _NL)
    cols = [jnp.full((_NL,), d, jnp.int32) for d in range(EMB_DIM)]

    def body(g, carry):
      iv = idx_v[pl.ds(g * _NL, _NL)]
      row_ids = lane + g * _NL
      for d in range(EMB_DIM):
        vals = plsc.load_gather(table_v, [iv, cols[d]])
        plsc.store_scatter(rows_v, [row_ids, cols[d]], vals)
      return carry

    lax.fori_loop(0, _GROUPS, body, 0)
    pltpu.sync_copy(rows_v, out_hbm.at[pl.ds(base, _B_PER_W)])

  return lookup


_lookup = _make_lookup()


@jax.jit
def kernel(x, embedding_table):
  idx = x.reshape(FLAT).astype(jnp.int32)
  out = _lookup(embedding_table, idx)
  return out.reshape(BATCH, PEP_LEN, EMB_DIM)


# speedup vs baseline: 4.6693x; 1.0023x over previous
import functools

import jax
import jax.numpy as jnp
from jax import lax
from jax.experimental import pallas as pl
from jax.experimental.pallas import tpu as pltpu
from jax.experimental.pallas import tpu_sc as plsc

NUM_ROWS = 20
EMB_DIM = 16
BATCH = 16384
PEP_LEN = 9
FLAT = BATCH * PEP_LEN

_info = plsc.get_sparse_core_info()
_NC, _NS, _NL = _info.num_cores, _info.num_subcores, _info.num_lanes
_NW = _NC * _NS
_B_PER_W = FLAT // _NW
_P_PER_W = BATCH // _NW
_GROUPS = _B_PER_W // _NL


def _make_lookup():
  mesh = plsc.VectorSubcoreMesh(core_axis_name="c", subcore_axis_name="s")

  @functools.partial(
      pl.kernel,
      mesh=mesh,
      out_type=jax.ShapeDtypeStruct((BATCH, PEP_LEN, EMB_DIM), jnp.float32),
      scratch_types=[
          pltpu.VMEM((NUM_ROWS, EMB_DIM), jnp.float32),
          pltpu.VMEM((_B_PER_W,), jnp.int32),
          pltpu.VMEM((_P_PER_W, PEP_LEN, EMB_DIM), jnp.float32),
      ],
      compiler_params=pltpu.CompilerParams(
          use_tc_tiling_on_sc=False, needs_layout_passes=False
      ),
  )
  def lookup(table_hbm, idx_hbm, out_hbm, table_v, idx_v, rows_v):
    wid = lax.axis_index("s") * _NC + lax.axis_index("c")
    base = wid * _B_PER_W
    pltpu.sync_copy(table_hbm, table_v)
    pltpu.sync_copy(idx_hbm.at[pl.ds(base, _B_PER_W)], idx_v)

    lane = lax.iota(jnp.int32, _NL)
    cols = [jnp.full((_NL,), d, jnp.int32) for d in range(EMB_DIM)]

    def body(g, carry):
      iv = idx_v[pl.ds(g * _NL, _NL)]
      r = lane + g * _NL
      p = r // PEP_LEN
      q = r - p * PEP_LEN
      for d in range(EMB_DIM):
        vals = plsc.load_gather(table_v, [iv, cols[d]])
        plsc.store_scatter(rows_v, [p, q, cols[d]], vals)
      return carry

    lax.fori_loop(0, _GROUPS, body, 0)
    pltpu.sync_copy(rows_v, out_hbm.at[pl.ds(wid * _P_PER_W, _P_PER_W)])

  return lookup


_lookup = _make_lookup()


@jax.jit
def kernel(x, embedding_table):
  idx = x.reshape(FLAT).astype(jnp.int32)
  return _lookup(embedding_table, idx)


# speedup vs baseline: 14.9441x; 3.2005x over previous
import functools

import jax
import jax.numpy as jnp
from jax import lax
from jax.experimental import pallas as pl
from jax.experimental.pallas import tpu as pltpu
from jax.experimental.pallas import tpu_sc as plsc

NUM_ROWS = 20
EMB_DIM = 16
BATCH = 16384
PEP_LEN = 9

_info = plsc.get_sparse_core_info()
_NC, _NS, _NL = _info.num_cores, _info.num_subcores, _info.num_lanes
_NW = _NC * _NS
_B_PER_W = BATCH // _NW
_GROUPS = _B_PER_W // _NL


def _make_lookup():
  mesh = plsc.VectorSubcoreMesh(core_axis_name="c", subcore_axis_name="s")

  @functools.partial(
      pl.kernel,
      mesh=mesh,
      out_type=jax.ShapeDtypeStruct((PEP_LEN, EMB_DIM, BATCH), jnp.float32),
      scratch_types=[
          pltpu.VMEM((EMB_DIM, NUM_ROWS), jnp.float32),
          pltpu.VMEM((PEP_LEN, _B_PER_W), jnp.int32),
          pltpu.VMEM((PEP_LEN, EMB_DIM, _B_PER_W), jnp.float32),
      ],
      compiler_params=pltpu.CompilerParams(
          use_tc_tiling_on_sc=False, needs_layout_passes=False
      ),
  )
  def lookup(table_hbm, idx_hbm, out_hbm, table_v, idx_v, rows_v):
    wid = lax.axis_index("s") * _NC + lax.axis_index("c")
    base = wid * _B_PER_W
    pltpu.sync_copy(table_hbm, table_v)
    pltpu.sync_copy(idx_hbm.at[:, pl.ds(base, _B_PER_W)], idx_v)

    cols = [jnp.full((_NL,), d, jnp.int32) for d in range(EMB_DIM)]

    def body(g, carry):
      for p in range(PEP_LEN):
        iv = idx_v[p, pl.ds(g * _NL, _NL)]
        for d in range(EMB_DIM):
          rows_v[p, d, pl.ds(g * _NL, _NL)] = plsc.load_gather(
              table_v, [cols[d], iv]
          )
      return carry

    lax.fori_loop(0, _GROUPS, body, 0)
    pltpu.sync_copy(rows_v, out_hbm.at[:, :, pl.ds(base, _B_PER_W)])

  return lookup


_lookup = _make_lookup()


@jax.jit
def kernel(x, embedding_table):
  xt = x.T.astype(jnp.int32)
  tt = embedding_table.T
  out_t = _lookup(tt, xt)
  return out_t.transpose(2, 0, 1)


# speedup vs baseline: 19.8421x; 1.3278x over previous
import functools

import jax
import jax.numpy as jnp
from jax import lax
from jax.experimental import pallas as pl
from jax.experimental.pallas import tpu as pltpu
from jax.experimental.pallas import tpu_sc as plsc

NUM_ROWS = 20
EMB_DIM = 16
BATCH = 16384
PEP_LEN = 9

_info = plsc.get_sparse_core_info()
_NC, _NS, _NL = _info.num_cores, _info.num_subcores, _info.num_lanes
_NW = _NC * _NS
_B_PER_W = BATCH // _NW
_GROUPS = _B_PER_W // _NL
_LT = BATCH // 128
_LT_PER_W = _B_PER_W // 128


def _make_lookup():
  mesh = plsc.VectorSubcoreMesh(core_axis_name="c", subcore_axis_name="s")

  @functools.partial(
      pl.kernel,
      mesh=mesh,
      out_type=jax.ShapeDtypeStruct((PEP_LEN, 2, _LT, 8, 128), jnp.float32),
      scratch_types=[
          pltpu.VMEM((EMB_DIM, NUM_ROWS), jnp.float32),
          pltpu.VMEM((PEP_LEN, _B_PER_W), jnp.int32),
          pltpu.VMEM((PEP_LEN, 2, _LT_PER_W, 8, 128), jnp.float32),
      ],
      compiler_params=pltpu.CompilerParams(
          use_tc_tiling_on_sc=False, needs_layout_passes=False
      ),
  )
  def lookup(table_hbm, idx_hbm, out_hbm, table_v, idx_v, rows_v):
    wid = lax.axis_index("s") * _NC + lax.axis_index("c")
    base = wid * _B_PER_W
    pltpu.sync_copy(table_hbm, table_v)
    pltpu.sync_copy(idx_hbm.at[:, pl.ds(base, _B_PER_W)], idx_v)

    cols = [jnp.full((_NL,), d, jnp.int32) for d in range(EMB_DIM)]

    def body(g, carry):
      ct = g // 8
      l0 = (g % 8) * _NL
      for p in range(PEP_LEN):
        iv = idx_v[p, pl.ds(g * _NL, _NL)]
        for d in range(EMB_DIM):
          rows_v[p, d // 8, ct, d % 8, pl.ds(l0, _NL)] = plsc.load_gather(
              table_v, [cols[d], iv]
          )
      return carry

    lax.fori_loop(0, _GROUPS, body, 0)
    pltpu.sync_copy(
        rows_v, out_hbm.at[:, :, pl.ds(wid * _LT_PER_W, _LT_PER_W)]
    )

  return lookup


_lookup = _make_lookup()


@jax.jit
def kernel(x, embedding_table):
  xt = x.T.astype(jnp.int32)
  tt = embedding_table.T
  z = _lookup(tt, xt)
  out_t = z.transpose(0, 1, 3, 2, 4).reshape(PEP_LEN, EMB_DIM, BATCH)
  return out_t.transpose(2, 0, 1)
